# baseline (speedup 1.0000x reference)
import jax
import jax.numpy as jnp
from jax.experimental import pallas as pl
from jax.experimental.pallas import tpu as pltpu

_TILE = 128
_NG = 8
_UNROLL = 4


def _pairwise_body(len_ref, x_ref, t_ref, out_ref, s_ref, ls_ref, tm_ref):
    x = x_ref[:, :]
    ls = -(jnp.maximum(-x, 0.0) + jnp.log1p(jnp.exp(-jnp.abs(x))))
    ls_ref[:, :] = ls
    s_ref[:, :] = jnp.exp(ls)

    iota_j1 = jax.lax.broadcasted_iota(jnp.int32, (1, _TILE), 1)
    ones_row = jnp.ones((1, _TILE), jnp.float32)
    nanf = jnp.float32(jnp.nan)
    zerof = jnp.float32(0.0)
    last_row = jnp.int32(_TILE - 1)

    def outer(v):
        return jax.lax.dot_general(
            v, ones_row, (((0,), (0,)), ((), ())),
            preferred_element_type=jnp.float32)

    def colsum(lhs, m):
        return jax.lax.dot_general(
            lhs, m, (((1,), (0,)), ((), ())),
            preferred_element_type=jnp.float32)

    acc = jnp.zeros((1, _TILE), jnp.float32)
    cnt = jnp.zeros((1, _TILE), jnp.float32)
    off = jnp.int32(0)
    for g in range(_NG):
        end = off + len_ref[g]
        lo = off // _TILE
        hi = (end + _TILE - 1) // _TILE
        off_g = off

        def mask_body(r, _, off=off_g, end=end):
            gi = iota_j1 + r * _TILE
            trow = t_ref[pl.ds(r, 1), :]
            tm_ref[pl.ds(r, 1), :] = jnp.where(
                (gi >= off) & (gi < end), trow, nanf)
            return 0
        jax.lax.fori_loop(lo, hi, mask_body, 0)

        def ti_body(ri, carry, lo=lo):
            acc1, cnt1 = carry
            si_row = s_ref[pl.ds(ri, 1), :]
            lsi_row = ls_ref[pl.ds(ri, 1), :]
            ti_row = tm_ref[pl.ds(ri, 1), :]
            s_i = outer(si_row)
            t_i = outer(ti_row)
            lhs2 = jnp.concatenate([ones_row, lsi_row], axis=0)

            def tile(rj, scale, acc2, cnt2):
                rjc = jnp.minimum(rj, last_row)
                sj_row = s_ref[pl.ds(rjc, 1), :]
                lsj_row = ls_ref[pl.ds(rjc, 1), :]
                tj_row = tm_ref[pl.ds(rjc, 1), :]
                p = jnp.log(s_i + sj_row)
                m1 = jnp.where(t_i > tj_row, scale, zerof)
                m2 = jnp.where(tj_row > t_i, scale, zerof)
                cf = m1 + m2
                cfp = cf * p
                bd = colsum(lhs2, m1)
                a = colsum(ones_row, cfp)
                c = colsum(ones_row, m2)
                acc2 = acc2 + (a - bd[1:2, :] - c * lsj_row)
                cnt2 = cnt2 + (bd[0:1, :] + c)
                return acc2, cnt2

            def tj_body(k, carry2, lo=lo):
                acc2, cnt2 = carry2
                rj0 = lo + _UNROLL * k
                for u in range(_UNROLL):
                    rj = rj0 + u
                    scale = jnp.where(
                        rj < ri, 1.0, jnp.where(rj == ri, 0.5, 0.0)
                    ).astype(jnp.float32)
                    acc2, cnt2 = tile(rj, scale, acc2, cnt2)
                return acc2, cnt2

            ntiles = ri - lo + 1
            return jax.lax.fori_loop(
                0, (ntiles + _UNROLL - 1) // _UNROLL, tj_body, (acc1, cnt1))

        acc, cnt = jax.lax.fori_loop(lo, hi, ti_body, (acc, cnt))
        off = end

    total = jnp.sum(acc)
    count = jnp.sum(cnt)
    out_ref[0, 0] = jnp.where(count > 0, total / count, 0.0)


def kernel(logits, targets, lengths):
    x2d = logits.reshape(_TILE, _TILE)
    t2d = targets.reshape(_TILE, _TILE)
    out = pl.pallas_call(
        _pairwise_body,
        out_shape=jax.ShapeDtypeStruct((1, 1), jnp.float32),
        in_specs=[
            pl.BlockSpec(memory_space=pltpu.SMEM),
            pl.BlockSpec(memory_space=pltpu.VMEM),
            pl.BlockSpec(memory_space=pltpu.VMEM),
        ],
        out_specs=pl.BlockSpec(memory_space=pltpu.SMEM),
        scratch_shapes=[
            pltpu.VMEM((_TILE, _TILE), jnp.float32),
            pltpu.VMEM((_TILE, _TILE), jnp.float32),
            pltpu.VMEM((_TILE, _TILE), jnp.float32),
        ],
    )(lengths, x2d, t2d)
    return out[0, 0]


# speedup vs baseline: 1.4793x; 1.4793x over previous
import jax
import jax.numpy as jnp
from jax.experimental import pallas as pl
from jax.experimental.pallas import tpu as pltpu

_TILE = 128
_NG = 8


def _pairwise_body(len_ref, x_ref, t_ref, out_ref, s_ref, ls_ref, tm_ref):
    x = x_ref[:, :]
    ls = -(jnp.maximum(-x, 0.0) + jnp.log1p(jnp.exp(-jnp.abs(x))))
    ls_ref[:, :] = ls
    s_ref[:, :] = jnp.exp(ls)

    iota_j1 = jax.lax.broadcasted_iota(jnp.int32, (1, _TILE), 1)
    ones_row = jnp.ones((1, _TILE), jnp.float32)
    onef = jnp.float32(1.0)
    zerof = jnp.float32(0.0)
    nanf = jnp.float32(jnp.nan)

    def outer(v):
        return jax.lax.dot_general(
            v, ones_row, (((0,), (0,)), ((), ())),
            preferred_element_type=jnp.float32)

    def fold(v):
        return jnp.sum(v.reshape(16, 8, _TILE), axis=0)

    acc = jnp.zeros((8, _TILE), jnp.float32)
    cnt = jnp.zeros((8, _TILE), jnp.float32)
    off = jnp.int32(0)
    for g in range(_NG):
        end = off + len_ref[g]
        lo = off // _TILE
        hi = (end + _TILE - 1) // _TILE
        off_g = off

        def mask_body(r, _, off=off_g, end=end):
            gi = iota_j1 + r * _TILE
            trow = t_ref[pl.ds(r, 1), :]
            tm_ref[pl.ds(r, 1), :] = jnp.where(
                (gi >= off) & (gi < end), trow, nanf)
            return 0
        jax.lax.fori_loop(lo, hi, mask_body, 0)

        def ti_body(ri, carry, lo=lo):
            acc1, cnt1 = carry
            si_row = s_ref[pl.ds(ri, 1), :]
            lsi_row = ls_ref[pl.ds(ri, 1), :]
            ti_row = tm_ref[pl.ds(ri, 1), :]
            s_i = outer(si_row)
            ls_i = outer(lsi_row)
            t_i = outer(ti_row)

            def tile(rj, scalef, acc2, cnt2):
                sj_row = s_ref[pl.ds(rj, 1), :]
                lsj_row = ls_ref[pl.ds(rj, 1), :]
                tj_row = tm_ref[pl.ds(rj, 1), :]
                p = jnp.log(s_i + sj_row)
                m1f = jnp.where(t_i > tj_row, scalef, zerof)
                m2f = jnp.where(tj_row > t_i, scalef, zerof)
                cf = m1f + m2f
                contrib = cf * p - m1f * ls_i - m2f * lsj_row
                return acc2 + fold(contrib), cnt2 + fold(cf)

            def diag_tile(rj, acc2, cnt2):
                sj_row = s_ref[pl.ds(rj, 1), :]
                tj_row = tm_ref[pl.ds(rj, 1), :]
                p = jnp.log(s_i + sj_row)
                m1f = jnp.where(t_i > tj_row, onef, zerof)
                return acc2 + fold(m1f * (p - ls_i)), cnt2 + fold(m1f)

            acc1, cnt1 = diag_tile(ri, acc1, cnt1)

            def tj_body(k, carry2):
                acc2, cnt2 = carry2
                rj = lo + 2 * k
                acc2, cnt2 = tile(rj, onef, acc2, cnt2)
                scale2 = jnp.where(rj + 1 < ri, onef, zerof)
                acc2, cnt2 = tile(rj + 1, scale2, acc2, cnt2)
                return acc2, cnt2

            npairs = ri - lo
            return jax.lax.fori_loop(
                0, (npairs + 1) // 2, tj_body, (acc1, cnt1))

        acc, cnt = jax.lax.fori_loop(lo, hi, ti_body, (acc, cnt))
        off = end

    total = jnp.sum(acc)
    count = jnp.sum(cnt.astype(jnp.int32))
    out_ref[0, 0] = jnp.where(
        count > 0, total / count.astype(jnp.float32), 0.0)


def kernel(logits, targets, lengths):
    x2d = logits.reshape(_TILE, _TILE)
    t2d = targets.reshape(_TILE, _TILE)
    out = pl.pallas_call(
        _pairwise_body,
        out_shape=jax.ShapeDtypeStruct((1, 1), jnp.float32),
        in_specs=[
            pl.BlockSpec(memory_space=pltpu.SMEM),
            pl.BlockSpec(memory_space=pltpu.VMEM),
            pl.BlockSpec(memory_space=pltpu.VMEM),
        ],
        out_specs=pl.BlockSpec(memory_space=pltpu.SMEM),
        scratch_shapes=[
            pltpu.VMEM((_TILE, _TILE), jnp.float32),
            pltpu.VMEM((_TILE, _TILE), jnp.float32),
            pltpu.VMEM((_TILE, _TILE), jnp.float32),
        ],
    )(lengths, x2d, t2d)
    return out[0, 0]


# speedup vs baseline: 1.5090x; 1.0201x over previous
import jax
import jax.numpy as jnp
from jax.experimental import pallas as pl
from jax.experimental.pallas import tpu as pltpu

_TILE = 128
_NG = 8


def _pairwise_body(len_ref, x_ref, t_ref, out_ref, s_ref, ls_ref, tm_ref):
    x = x_ref[:, :]
    ls = -(jnp.maximum(-x, 0.0) + jnp.log1p(jnp.exp(-jnp.abs(x))))
    ls_ref[:, :] = ls
    s_ref[:, :] = jnp.exp(ls)

    iota_j1 = jax.lax.broadcasted_iota(jnp.int32, (1, _TILE), 1)
    ones_row = jnp.ones((1, _TILE), jnp.float32)
    onef = jnp.float32(1.0)
    zerof = jnp.float32(0.0)
    nanf = jnp.float32(jnp.nan)

    def outer(v):
        return jax.lax.dot_general(
            v, ones_row, (((0,), (0,)), ((), ())),
            preferred_element_type=jnp.float32)

    def fold(v):
        return jnp.sum(v.reshape(16, 8, _TILE), axis=0)

    acc = jnp.zeros((8, _TILE), jnp.float32)
    cnt = jnp.zeros((8, _TILE), jnp.float32)
    off = jnp.int32(0)
    for g in range(_NG):
        end = off + len_ref[g]
        lo = off // _TILE
        hi = (end + _TILE - 1) // _TILE
        off_g = off

        def mask_body(r, _, off=off_g, end=end):
            gi = iota_j1 + r * _TILE
            trow = t_ref[pl.ds(r, 1), :]
            tm_ref[pl.ds(r, 1), :] = jnp.where(
                (gi >= off) & (gi < end), trow, nanf)
            return 0
        jax.lax.fori_loop(lo, hi, mask_body, 0)

        def ti_body(ri, carry, lo=lo):
            acc1, cnt1 = carry
            si_row = s_ref[pl.ds(ri, 1), :]
            lsi_row = ls_ref[pl.ds(ri, 1), :]
            ti_row = tm_ref[pl.ds(ri, 1), :]
            s_i = outer(si_row)
            ls_i = outer(lsi_row)
            t_i = outer(ti_row)

            def tile(rj, live, acc2, cnt2):
                sj_row = s_ref[pl.ds(rj, 1), :]
                lsj_row = ls_ref[pl.ds(rj, 1), :]
                tj_row = jnp.where(live, tm_ref[pl.ds(rj, 1), :], nanf)
                p = jnp.log(s_i + sj_row)
                c1 = t_i > tj_row
                c2 = tj_row > t_i
                cf = jnp.where(c1 | c2, onef, zerof)
                lssel = jnp.where(c1, ls_i, jnp.where(c2, lsj_row, zerof))
                return acc2 + fold(cf * p - lssel), cnt2 + fold(cf)

            def diag_tile(rj, acc2, cnt2):
                sj_row = s_ref[pl.ds(rj, 1), :]
                tj_row = tm_ref[pl.ds(rj, 1), :]
                p = jnp.log(s_i + sj_row)
                c1 = t_i > tj_row
                contrib = jnp.where(c1, p - ls_i, zerof)
                return acc2 + fold(contrib), cnt2 + fold(
                    jnp.where(c1, onef, zerof))

            acc1, cnt1 = diag_tile(ri, acc1, cnt1)

            def tj_body(k, carry2):
                acc2, cnt2 = carry2
                rj = lo + 2 * k
                acc2, cnt2 = tile(rj, True, acc2, cnt2)
                acc2, cnt2 = tile(rj + 1, rj + 1 < ri, acc2, cnt2)
                return acc2, cnt2

            npairs = ri - lo
            return jax.lax.fori_loop(
                0, (npairs + 1) // 2, tj_body, (acc1, cnt1))

        acc, cnt = jax.lax.fori_loop(lo, hi, ti_body, (acc, cnt))
        off = end

    total = jnp.sum(acc)
    count = jnp.sum(cnt.astype(jnp.int32))
    out_ref[0, 0] = jnp.where(
        count > 0, total / count.astype(jnp.float32), 0.0)


def kernel(logits, targets, lengths):
    x2d = logits.reshape(_TILE, _TILE)
    t2d = targets.reshape(_TILE, _TILE)
    out = pl.pallas_call(
        _pairwise_body,
        out_shape=jax.ShapeDtypeStruct((1, 1), jnp.float32),
        in_specs=[
            pl.BlockSpec(memory_space=pltpu.SMEM),
            pl.BlockSpec(memory_space=pltpu.VMEM),
            pl.BlockSpec(memory_space=pltpu.VMEM),
        ],
        out_specs=pl.BlockSpec(memory_space=pltpu.SMEM),
        scratch_shapes=[
            pltpu.VMEM((_TILE, _TILE), jnp.float32),
            pltpu.VMEM((_TILE, _TILE), jnp.float32),
            pltpu.VMEM((_TILE, _TILE), jnp.float32),
        ],
    )(lengths, x2d, t2d)
    return out[0, 0]
